# baseline (speedup 1.0000x reference)
import functools

import jax
import jax.numpy as jnp
from jax.experimental import pallas as pl
from jax.experimental.pallas import tpu as pltpu

EMB_D = 64
K = 512
G = 32


def _vq_kernel(z_ref, e_ref, o_ref):
    e = e_ref[...]
    es = e * -2.0
    e2 = jnp.sum(e * e, axis=1, keepdims=True)
    e_hi = e.astype(jnp.bfloat16)
    e_lo = (e - e_hi.astype(jnp.float32)).astype(jnp.bfloat16)
    e_cat = jnp.concatenate([e_hi, e_lo], axis=1)
    for g in range(G):
        z = z_ref[g]
        d = e2 + jax.lax.dot_general(
            es, z, (((1,), (0,)), ((), ())),
            preferred_element_type=jnp.float32,
        )
        m = jnp.min(d, axis=0, keepdims=True)
        onehot = (1.0 + jnp.sign(m - d)).astype(jnp.bfloat16)
        r = jax.lax.dot_general(
            e_cat, onehot, (((0,), (0,)), ((), ())),
            preferred_element_type=jnp.float32,
        )
        o_ref[g] = r[:EMB_D] + r[EMB_D:]


@jax.jit
def kernel(z_e, e):
    B, C, H, W = z_e.shape
    HW = H * W
    z = z_e.reshape(B, C, HW)
    out = pl.pallas_call(
        _vq_kernel,
        grid=(B // G,),
        in_specs=[
            pl.BlockSpec((G, C, HW), lambda i: (i, 0, 0)),
            pl.BlockSpec((K, EMB_D), lambda i: (0, 0)),
        ],
        out_specs=pl.BlockSpec((G, C, HW), lambda i: (i, 0, 0)),
        out_shape=jax.ShapeDtypeStruct((B, C, HW), jnp.float32),
        compiler_params=pltpu.CompilerParams(
            dimension_semantics=("parallel",),
        ),
    )(z, e)
    return out.reshape(B, C, H, W)


# speedup vs baseline: 1.1786x; 1.1786x over previous
import functools

import jax
import jax.numpy as jnp
from jax.experimental import pallas as pl
from jax.experimental.pallas import tpu as pltpu

EMB_D = 64
K = 512
G = 32


def _vq_kernel(z_ref, e_ref, o_ref):
    e = e_ref[...]
    es = e * -2.0
    e2 = jnp.sum(e * e, axis=1, keepdims=True)
    e_hi = e.astype(jnp.bfloat16)
    e_lo = (e - e_hi.astype(jnp.float32)).astype(jnp.bfloat16)
    e_cat = jnp.concatenate([e_hi, e_lo], axis=1)
    for g in range(G):
        z = z_ref[g]
        d = e2 + jax.lax.dot_general(
            es, z, (((1,), (0,)), ((), ())),
            preferred_element_type=jnp.float32,
        )
        m = jnp.min(d, axis=0, keepdims=True)
        onehot = (d == m).astype(jnp.bfloat16)
        r = jax.lax.dot_general(
            e_cat, onehot, (((0,), (0,)), ((), ())),
            preferred_element_type=jnp.float32,
        )
        o_ref[g] = r[:EMB_D] + r[EMB_D:]


@jax.jit
def kernel(z_e, e):
    B, C, H, W = z_e.shape
    HW = H * W
    z = z_e.reshape(B, C, HW)
    out = pl.pallas_call(
        _vq_kernel,
        grid=(B // G,),
        in_specs=[
            pl.BlockSpec((G, C, HW), lambda i: (i, 0, 0)),
            pl.BlockSpec((K, EMB_D), lambda i: (0, 0)),
        ],
        out_specs=pl.BlockSpec((G, C, HW), lambda i: (i, 0, 0)),
        out_shape=jax.ShapeDtypeStruct((B, C, HW), jnp.float32),
        compiler_params=pltpu.CompilerParams(
            dimension_semantics=("parallel",),
        ),
    )(z, e)
    return out.reshape(B, C, H, W)


# speedup vs baseline: 1.8757x; 1.5915x over previous
import functools

import jax
import jax.numpy as jnp
from jax.experimental import pallas as pl
from jax.experimental.pallas import tpu as pltpu

EMB_D = 64
K = 512
G = 64


def _vq_kernel(z_ref, e_ref, o_ref):
    e = e_ref[...]
    es = e * -2.0
    e2 = jnp.sum(e * e, axis=1, keepdims=True)
    e_hi = e.astype(jnp.bfloat16)
    e_lo = (e - e_hi.astype(jnp.float32)).astype(jnp.bfloat16)
    for g in range(G):
        z = z_ref[g]
        d = e2 + jax.lax.dot_general(
            es, z, (((1,), (0,)), ((), ())),
            preferred_element_type=jnp.float32,
        )
        m = jnp.min(d, axis=0, keepdims=True)
        onehot = (d == m).astype(jnp.bfloat16)
        o_ref[g] = jax.lax.dot_general(
            e_hi, onehot, (((0,), (0,)), ((), ())),
            preferred_element_type=jnp.float32,
        ) + jax.lax.dot_general(
            e_lo, onehot, (((0,), (0,)), ((), ())),
            preferred_element_type=jnp.float32,
        )


@jax.jit
def kernel(z_e, e):
    B, C, H, W = z_e.shape
    HW = H * W
    z = z_e.reshape(B, C, HW)
    out = pl.pallas_call(
        _vq_kernel,
        grid=(B // G,),
        in_specs=[
            pl.BlockSpec((G, C, HW), lambda i: (i, 0, 0)),
            pl.BlockSpec((K, EMB_D), lambda i: (0, 0)),
        ],
        out_specs=pl.BlockSpec((G, C, HW), lambda i: (i, 0, 0)),
        out_shape=jax.ShapeDtypeStruct((B, C, HW), jnp.float32),
        compiler_params=pltpu.CompilerParams(
            dimension_semantics=("parallel",),
        ),
    )(z, e)
    return out.reshape(B, C, H, W)
